# Initial kernel scaffold; baseline (speedup 1.0000x reference)
import jax
import jax.numpy as jnp
from jax.experimental import pallas as pl


def kernel(x, adj_indices, adj_values, W, b):
    raise NotImplementedError("write your pallas kernel here")


# speedup vs baseline: 4.1920x; 4.1920x over previous
import functools

import jax
import jax.numpy as jnp
from jax import lax
from jax.experimental import pallas as pl
from jax.experimental.pallas import tpu as pltpu
from jax.experimental.pallas import tpu_sc as plsc

N = 10000
D = 128
E = 320000

NC = 2
NS = 16
NW = NC * NS
CHUNK = 128
CHUNKS_PER_TILE = -(-E // (CHUNK * NW))
E_PAD = CHUNK * NW * CHUNKS_PER_TILE
ROWS_PER_TILE = N // NS


def _linear(x, W, b):
    def body(x_ref, w_ref, b_ref, o_ref):
        o_ref[...] = lax.dot_general(
            x_ref[...], w_ref[...], (((1,), (1,)), ((), ())),
            preferred_element_type=jnp.float32,
            precision=lax.Precision.HIGHEST,
        ) + b_ref[...]

    return pl.pallas_call(
        body,
        out_shape=jax.ShapeDtypeStruct((N, D), jnp.float32),
    )(x, W, b.reshape(1, D))


def _add_partials(p):
    def body(p_ref, o_ref):
        o_ref[...] = p_ref[0] + p_ref[1]

    return pl.pallas_call(
        body,
        out_shape=jax.ShapeDtypeStruct((N, D), jnp.float32),
    )(p)


@functools.partial(
    pl.kernel,
    out_type=jax.ShapeDtypeStruct((NC, N, D), jnp.float32),
    mesh=plsc.VectorSubcoreMesh(core_axis_name="c", subcore_axis_name="s"),
    scratch_types=[
        pltpu.VMEM((2, CHUNK), jnp.int32),
        pltpu.VMEM((CHUNK,), jnp.float32),
        pltpu.VMEM((CHUNK, D), jnp.float32),
        pltpu.VMEM_SHARED((N, D), jnp.float32),
        pltpu.SemaphoreType.DMA,
    ],
)
def _sc_aggregate(support_hbm, idx_hbm, val_hbm, out_hbm,
                  idx_v, val_v, rows_v, acc, sem):
    cid = lax.axis_index("c")
    tid = lax.axis_index("s")
    wid = tid * NC + cid

    @pl.loop(0, CHUNK)
    def _(g):
        r = rows_v.at[g]
        for d in range(D // 16):
            r[pl.ds(d * 16, 16)] = jnp.zeros((16,), jnp.float32)

    base = tid * ROWS_PER_TILE
    for j in range(5):
        pltpu.sync_copy(rows_v.at[pl.ds(0, 125)],
                        acc.at[pl.ds(base + j * 125, 125)])
    plsc.subcore_barrier()

    @pl.loop(0, CHUNKS_PER_TILE)
    def _(k):
        c = wid * CHUNKS_PER_TILE + k
        pltpu.sync_copy(idx_hbm.at[c], idx_v)
        pltpu.sync_copy(val_hbm.at[c], val_v)
        pltpu.async_copy(support_hbm.at[idx_v.at[1]], rows_v, sem).wait()

        @pl.loop(0, CHUNK // 16)
        def _(j):
            vals16 = val_v[pl.ds(j * 16, 16)]
            for g in range(16):
                v = vals16[g]
                r = rows_v.at[j * 16 + g]
                for d in range(D // 16):
                    sl = pl.ds(d * 16, 16)
                    r[sl] = r[sl] * v

        pltpu.sync_copy(rows_v, acc.at[idx_v.at[0]], add=True)

    plsc.subcore_barrier()
    wb = tid * 624
    pltpu.sync_copy(acc.at[pl.ds(wb, 624)],
                    out_hbm.at[cid, pl.ds(wb, 624)])

    @pl.when(tid == 0)
    def _():
        pltpu.sync_copy(acc.at[pl.ds(16 * 624, N - 16 * 624)],
                        out_hbm.at[cid, pl.ds(16 * 624, N - 16 * 624)])


@jax.jit
def kernel(x, adj_indices, adj_values, W, b):
    support = _linear(x, W, b)

    pad = E_PAD - E
    row = adj_indices[0]
    col = adj_indices[1]
    idx = jnp.stack([
        jnp.pad(row, (0, pad)).reshape(NW * CHUNKS_PER_TILE, CHUNK),
        jnp.pad(col, (0, pad)).reshape(NW * CHUNKS_PER_TILE, CHUNK),
    ], axis=1)
    vals = jnp.pad(adj_values, (0, pad)).reshape(NW * CHUNKS_PER_TILE, CHUNK)

    partials = _sc_aggregate(support, idx, vals)
    return _add_partials(partials)
